# baseline (speedup 1.0000x reference)
import functools

import jax
import jax.numpy as jnp
from jax.experimental import pallas as pl
from jax.experimental.pallas import tpu as pltpu


def _body(q_ref, k_ref, score_ref, idx_ref, *, block_k: int, total_k: int):
    i = pl.program_id(0)
    q = q_ref[...]
    k = k_ref[...]

    dots = jax.lax.dot_general(
        q, k, (((1,), (1,)), ((), ())),
        preferred_element_type=jnp.float32,
    )

    q_norm = jnp.sqrt(jnp.sum(q * q, axis=1, keepdims=True))
    k_norm = jnp.sqrt(jnp.sum(k * k, axis=1, keepdims=True))
    denom = q_norm * k_norm.reshape(1, -1)
    cos = dots / denom

    gid = i * block_k + jax.lax.broadcasted_iota(jnp.int32, cos.shape, 1)
    cos = jnp.where(gid < total_k, cos, -jnp.inf)

    blk_max = jnp.max(cos, axis=1, keepdims=True)
    blk_arg = jnp.min(
        jnp.where(cos == blk_max, gid, jnp.int32(2**31 - 1)),
        axis=1, keepdims=True,
    )

    @pl.when(i == 0)
    def _():
        score_ref[...] = blk_max
        idx_ref[...] = blk_arg

    @pl.when(i > 0)
    def _():
        prev = score_ref[...]
        better = blk_max > prev
        score_ref[...] = jnp.where(better, blk_max, prev)
        idx_ref[...] = jnp.where(better, blk_arg, idx_ref[...])


def kernel(queries, keys):
    b, d = queries.shape
    total_k = keys.shape[0]
    block_k = 1024
    nb = pl.cdiv(total_k, block_k)

    score, idx = pl.pallas_call(
        functools.partial(_body, block_k=block_k, total_k=total_k),
        grid=(nb,),
        in_specs=[
            pl.BlockSpec((b, d), lambda i: (0, 0)),
            pl.BlockSpec((block_k, d), lambda i: (i, 0)),
        ],
        out_specs=[
            pl.BlockSpec((b, 1), lambda i: (0, 0)),
            pl.BlockSpec((b, 1), lambda i: (0, 0)),
        ],
        out_shape=[
            jax.ShapeDtypeStruct((b, 1), jnp.float32),
            jax.ShapeDtypeStruct((b, 1), jnp.int32),
        ],
        compiler_params=pltpu.CompilerParams(
            dimension_semantics=("arbitrary",),
        ),
    )(queries, keys)
    return score.reshape(b), idx.reshape(b)


# speedup vs baseline: 1.0715x; 1.0715x over previous
import functools

import jax
import jax.numpy as jnp
from jax.experimental import pallas as pl
from jax.experimental.pallas import tpu as pltpu


def _score_block(q, k, base, total_k):
    dots = jax.lax.dot_general(
        q, k, (((1,), (1,)), ((), ())),
        preferred_element_type=jnp.float32,
    )
    q_norm = jnp.sqrt(jnp.sum(q * q, axis=1, keepdims=True))
    k_norm = jnp.sqrt(jnp.sum(k * k, axis=1, keepdims=True))
    cos = dots / (q_norm * k_norm.reshape(1, -1))
    gid = base + jax.lax.broadcasted_iota(jnp.int32, cos.shape, 1)
    cos = jnp.where(gid < total_k, cos, -jnp.inf)
    blk_max = jnp.max(cos, axis=1, keepdims=True)
    blk_arg = jnp.min(
        jnp.where(cos == blk_max, gid, jnp.int32(2**31 - 1)),
        axis=1, keepdims=True,
    )
    return blk_max, blk_arg


def _body(q_ref, ka_ref, kb_ref, score_ref, idx_ref, *,
          block_k: int, half_blocks: int, total_k: int):
    i = pl.program_id(0)
    q = q_ref[...]

    max_a, arg_a = _score_block(q, ka_ref[...], i * block_k, total_k)
    max_b, arg_b = _score_block(q, kb_ref[...], (half_blocks + i) * block_k,
                                total_k)

    blk_max = jnp.where(max_b > max_a, max_b, max_a)
    blk_arg = jnp.where(max_b > max_a, arg_b, arg_a)

    @pl.when(i == 0)
    def _():
        score_ref[...] = blk_max
        idx_ref[...] = blk_arg

    @pl.when(i > 0)
    def _():
        prev = score_ref[...]
        better = blk_max > prev
        score_ref[...] = jnp.where(better, blk_max, prev)
        idx_ref[...] = jnp.where(better, blk_arg, idx_ref[...])


def kernel(queries, keys):
    b, d = queries.shape
    total_k = keys.shape[0]
    block_k = 1024
    nb = pl.cdiv(total_k, block_k)
    half_blocks = (nb + 1) // 2

    score, idx = pl.pallas_call(
        functools.partial(_body, block_k=block_k, half_blocks=half_blocks,
                          total_k=total_k),
        grid=(half_blocks,),
        in_specs=[
            pl.BlockSpec((b, d), lambda i: (0, 0)),
            pl.BlockSpec((block_k, d), lambda i: (i, 0)),
            pl.BlockSpec((block_k, d),
                         lambda i, hb=half_blocks: (hb + i, 0)),
        ],
        out_specs=[
            pl.BlockSpec((b, 1), lambda i: (0, 0)),
            pl.BlockSpec((b, 1), lambda i: (0, 0)),
        ],
        out_shape=[
            jax.ShapeDtypeStruct((b, 1), jnp.float32),
            jax.ShapeDtypeStruct((b, 1), jnp.int32),
        ],
        compiler_params=pltpu.CompilerParams(
            dimension_semantics=("arbitrary",),
        ),
    )(queries, keys, keys)
    return score.reshape(b), idx.reshape(b)


# speedup vs baseline: 1.0753x; 1.0035x over previous
import functools

import jax
import jax.numpy as jnp
from jax.experimental import pallas as pl
from jax.experimental.pallas import tpu as pltpu


def _body(q_ref, k_ref, score_ref, idx_ref, *, block_k: int, total_k: int):
    i = pl.program_id(0)
    q = q_ref[...]
    k = k_ref[...]

    dots = jax.lax.dot_general(
        q, k, (((1,), (1,)), ((), ())),
        preferred_element_type=jnp.float32,
    )

    q_norm = jnp.sqrt(jnp.sum(q * q, axis=1, keepdims=True))
    k_norm = jnp.sqrt(jnp.sum(k * k, axis=1, keepdims=True))
    cos = dots / (q_norm * k_norm.reshape(1, -1))

    gid = i * block_k + jax.lax.broadcasted_iota(jnp.int32, cos.shape, 1)
    cos = jnp.where(gid < total_k, cos, -jnp.inf)

    blk_max = jnp.max(cos, axis=1, keepdims=True)
    blk_arg = jnp.min(
        jnp.where(cos == blk_max, gid, jnp.int32(2**31 - 1)),
        axis=1, keepdims=True,
    )

    @pl.when(i == 0)
    def _():
        score_ref[...] = blk_max
        idx_ref[...] = blk_arg

    @pl.when(i > 0)
    def _():
        prev = score_ref[...]
        better = blk_max > prev
        score_ref[...] = jnp.where(better, blk_max, prev)
        idx_ref[...] = jnp.where(better, blk_arg, idx_ref[...])


def kernel(queries, keys):
    b, d = queries.shape
    total_k = keys.shape[0]
    block_k = 2048
    nb = pl.cdiv(total_k, block_k)

    score, idx = pl.pallas_call(
        functools.partial(_body, block_k=block_k, total_k=total_k),
        grid=(nb,),
        in_specs=[
            pl.BlockSpec((b, d), lambda i: (0, 0)),
            pl.BlockSpec((block_k, d), lambda i: (i, 0)),
        ],
        out_specs=[
            pl.BlockSpec((b, 1), lambda i: (0, 0)),
            pl.BlockSpec((b, 1), lambda i: (0, 0)),
        ],
        out_shape=[
            jax.ShapeDtypeStruct((b, 1), jnp.float32),
            jax.ShapeDtypeStruct((b, 1), jnp.int32),
        ],
        compiler_params=pltpu.CompilerParams(
            dimension_semantics=("arbitrary",),
        ),
    )(queries, keys)
    return score.reshape(b), idx.reshape(b)
